# speedup vs baseline: 1.4879x; 1.4879x over previous
import jax
import jax.numpy as jnp
from jax.experimental import pallas as pl
from jax.experimental.pallas import tpu as pltpu


def _fused_kernel(bev_ref, sem_ref, com_ref,
                  wsT_ref, wcT_ref, wabT_ref, wasT_ref, wacT_ref,
                  bs_ref, bc_ref, bab_ref, bas_ref, bac_ref,
                  o_ref):
    HW = bev_ref.shape[2]
    inv_hw = jnp.float32(1.0 / HW)

    pb = jnp.sum(bev_ref[0], axis=1, keepdims=True) * inv_hw
    ps = jnp.sum(sem_ref[0], axis=1, keepdims=True) * inv_hw
    pc = jnp.sum(com_ref[0], axis=1, keepdims=True) * inv_hw

    ps_a = jnp.dot(wsT_ref[...], ps, preferred_element_type=jnp.float32) + bs_ref[...]
    pc_a = jnp.dot(wcT_ref[...], pc, preferred_element_type=jnp.float32) + bc_ref[...]
    a_b = jax.nn.sigmoid(
        jnp.dot(wabT_ref[...], pb, preferred_element_type=jnp.float32) + bab_ref[...])
    a_s = jax.nn.sigmoid(
        jnp.dot(wasT_ref[...], ps_a, preferred_element_type=jnp.float32) + bas_ref[...])
    a_c = jax.nn.sigmoid(
        jnp.dot(wacT_ref[...], pc_a, preferred_element_type=jnp.float32) + bac_ref[...])

    const = bs_ref[...] * a_s + bc_ref[...] * a_c

    chunk = 4096 if HW % 4096 == 0 else HW
    for i in range(HW // chunk):
        sl = pl.ds(i * chunk, chunk)
        sem_a = jnp.dot(wsT_ref[...], sem_ref[0, :, sl],
                        preferred_element_type=jnp.float32)
        com_a = jnp.dot(wcT_ref[...], com_ref[0, :, sl],
                        preferred_element_type=jnp.float32)
        o_ref[0, :, sl] = (bev_ref[0, :, sl] * a_b
                           + sem_a * a_s + com_a * a_c + const)


def kernel(bev, sem, com,
           w_adp_sem, b_adp_sem, w_adp_com, b_adp_com,
           w_att_bev, b_att_bev, w_att_sem, b_att_sem,
           w_att_com, b_att_com):
    N, C, H, W = bev.shape
    Ch = sem.shape[1]
    HW = H * W

    bev3 = bev.reshape(N, C, HW)
    sem3 = sem.reshape(N, Ch, HW)
    com3 = com.reshape(N, Ch, HW)

    wsT = w_adp_sem.T
    wcT = w_adp_com.T
    wabT = w_att_bev.T
    wasT = w_att_sem.T
    wacT = w_att_com.T
    col = lambda b: b.reshape(C, 1)

    out = pl.pallas_call(
        _fused_kernel,
        out_shape=jax.ShapeDtypeStruct((N, C, HW), jnp.float32),
        grid=(N,),
        in_specs=[
            pl.BlockSpec((1, C, HW), lambda n: (n, 0, 0)),
            pl.BlockSpec((1, Ch, HW), lambda n: (n, 0, 0)),
            pl.BlockSpec((1, Ch, HW), lambda n: (n, 0, 0)),
            pl.BlockSpec((C, Ch), lambda n: (0, 0)),
            pl.BlockSpec((C, Ch), lambda n: (0, 0)),
            pl.BlockSpec((C, C), lambda n: (0, 0)),
            pl.BlockSpec((C, C), lambda n: (0, 0)),
            pl.BlockSpec((C, C), lambda n: (0, 0)),
            pl.BlockSpec((C, 1), lambda n: (0, 0)),
            pl.BlockSpec((C, 1), lambda n: (0, 0)),
            pl.BlockSpec((C, 1), lambda n: (0, 0)),
            pl.BlockSpec((C, 1), lambda n: (0, 0)),
            pl.BlockSpec((C, 1), lambda n: (0, 0)),
        ],
        out_specs=pl.BlockSpec((1, C, HW), lambda n: (n, 0, 0)),
        compiler_params=pltpu.CompilerParams(
            dimension_semantics=("parallel",)),
    )(bev3, sem3, com3, wsT, wcT, wabT, wasT, wacT,
      col(b_adp_sem), col(b_adp_com),
      col(b_att_bev), col(b_att_sem), col(b_att_com))

    return out.reshape(N, C, H, W)


# speedup vs baseline: 3.2324x; 2.1725x over previous
import jax
import jax.numpy as jnp
from jax.experimental import pallas as pl
from jax.experimental.pallas import tpu as pltpu

_RH = 8
_RB = 32
_NJ = _RB // _RH


def _fused_kernel(bev_ref, sem_ref, com_ref,
                  wsT_ref, wcT_ref, wabT_ref, wasT_ref, wacT_ref,
                  bs_ref, bc_ref, bab_ref, bas_ref, bac_ref,
                  o_ref,
                  cbev_ref, csem_ref, ccom_ref,
                  accb_ref, accs_ref, accc_ref,
                  gb_ref, gs_ref, gc_ref, k_ref,
                  tsem_ref, tcom_ref, sout_ref):
    t = pl.program_id(1)
    nt = pl.num_programs(1) // 2
    C, RH, W = accb_ref.shape
    Ch = accs_ref.shape[0]
    S = Ch + _RH
    S8 = S * _RH
    CW = _RB * W

    @pl.when(t < nt)
    def _phase_a():
        cbev_ref[:, pl.ds(t * _RB, _RB), :] = bev_ref[0]
        for c in range(Ch):
            for j in range(_NJ):
                tsem_ref[j * S8 + c: j * S8 + c + S8: S, :] = \
                    sem_ref[0, c, j * _RH:(j + 1) * _RH, :]
                tcom_ref[j * S8 + c: j * S8 + c + S8: S, :] = \
                    com_ref[0, c, j * _RH:(j + 1) * _RH, :]
        sem_cat = jnp.concatenate(
            [tsem_ref[pl.ds(j * S8 + S * s, Ch), :]
             for j in range(_NJ) for s in range(_RH)], axis=1)
        com_cat = jnp.concatenate(
            [tcom_ref[pl.ds(j * S8 + S * s, Ch), :]
             for j in range(_NJ) for s in range(_RH)], axis=1)
        csem_ref[:, pl.ds(t * CW, CW)] = sem_cat
        ccom_ref[:, pl.ds(t * CW, CW)] = com_cat

        def tilesum(ref):
            v = ref[0]
            return sum(v[:, j * _RH:(j + 1) * _RH, :] for j in range(_NJ))

        @pl.when(t == 0)
        def _():
            accb_ref[...] = tilesum(bev_ref)
            accs_ref[...] = tilesum(sem_ref)
            accc_ref[...] = tilesum(com_ref)

        @pl.when(t != 0)
        def _():
            accb_ref[...] += tilesum(bev_ref)
            accs_ref[...] += tilesum(sem_ref)
            accc_ref[...] += tilesum(com_ref)

    @pl.when(t == nt)
    def _gates():
        inv_hw = jnp.float32(1.0 / (nt * _RB * W))

        def fold(acc_ref, n_ch):
            flat = acc_ref[...].reshape(n_ch * RH, W)
            r = jax.lax.broadcasted_iota(jnp.int32, (n_ch, n_ch * RH), 0)
            q = jax.lax.broadcasted_iota(jnp.int32, (n_ch, n_ch * RH), 1)
            sel = (q // RH == r).astype(jnp.float32)
            per_w = jnp.dot(sel, flat, preferred_element_type=jnp.float32)
            return jnp.sum(per_w, axis=1, keepdims=True)

        pb = fold(accb_ref, C) * inv_hw
        ps = fold(accs_ref, Ch) * inv_hw
        pc = fold(accc_ref, Ch) * inv_hw
        ps_a = jnp.dot(wsT_ref[...], ps, preferred_element_type=jnp.float32) + bs_ref[...]
        pc_a = jnp.dot(wcT_ref[...], pc, preferred_element_type=jnp.float32) + bc_ref[...]
        a_b = jax.nn.sigmoid(
            jnp.dot(wabT_ref[...], pb, preferred_element_type=jnp.float32) + bab_ref[...])
        a_s = jax.nn.sigmoid(
            jnp.dot(wasT_ref[...], ps_a, preferred_element_type=jnp.float32) + bas_ref[...])
        a_c = jax.nn.sigmoid(
            jnp.dot(wacT_ref[...], pc_a, preferred_element_type=jnp.float32) + bac_ref[...])
        konst = bs_ref[...] * a_s + bc_ref[...] * a_c
        gs_ref[...] = jnp.broadcast_to(a_s, (C, W))
        gc_ref[...] = jnp.broadcast_to(a_c, (C, W))
        k_ref[...] = jnp.broadcast_to(konst, (C, W))
        gb_w = jnp.broadcast_to(a_b, (C, W))
        for s in range(_RH):
            sout_ref[s: s + _RH * C: _RH, :] = gb_w
        gb_ref[...] = sout_ref[...].reshape(C, _RH, W)

    @pl.when(t >= nt)
    def _phase_b():
        tb = t - nt
        sem_a = jnp.dot(wsT_ref[...], csem_ref[:, pl.ds(tb * CW, CW)],
                        preferred_element_type=jnp.float32)
        com_a = jnp.dot(wcT_ref[...], ccom_ref[:, pl.ds(tb * CW, CW)],
                        preferred_element_type=jnp.float32)
        gs = gs_ref[...]
        gc = gc_ref[...]
        kk = k_ref[...]
        gbr = gb_ref[...]
        for j in range(_NJ):
            for s in range(_RH):
                k_col = (j * _RH + s) * W
                chunk = (sem_a[:, k_col:k_col + W] * gs
                         + com_a[:, k_col:k_col + W] * gc + kk)
                sout_ref[s: s + _RH * C: _RH, :] = chunk
            o_ref[0, :, j * _RH:(j + 1) * _RH, :] = (
                sout_ref[...].reshape(C, _RH, W)
                + cbev_ref[:, pl.ds(tb * _RB + j * _RH, _RH), :] * gbr)


def kernel(bev, sem, com,
           w_adp_sem, b_adp_sem, w_adp_com, b_adp_com,
           w_att_bev, b_att_bev, w_att_sem, b_att_sem,
           w_att_com, b_att_com):
    N, C, H, W = bev.shape
    Ch = sem.shape[1]
    nt = H // _RB
    S = Ch + _RH

    wsT = w_adp_sem.T
    wcT = w_adp_com.T
    wabT = w_att_bev.T
    wasT = w_att_sem.T
    wacT = w_att_com.T
    col = lambda b: b.reshape(C, 1)

    w_spec = lambda r, c_: pl.BlockSpec((r, c_), lambda n, t: (0, 0))
    last = nt - 1

    out = pl.pallas_call(
        _fused_kernel,
        out_shape=jax.ShapeDtypeStruct((N, C, H, W), jnp.float32),
        grid=(N, 2 * nt),
        in_specs=[
            pl.BlockSpec((1, C, _RB, W),
                         lambda n, t: (n, 0, jnp.where(t < nt, t, last), 0)),
            pl.BlockSpec((1, Ch, _RB, W),
                         lambda n, t: (n, 0, jnp.where(t < nt, t, last), 0)),
            pl.BlockSpec((1, Ch, _RB, W),
                         lambda n, t: (n, 0, jnp.where(t < nt, t, last), 0)),
            w_spec(C, Ch), w_spec(C, Ch),
            w_spec(C, C), w_spec(C, C), w_spec(C, C),
            w_spec(C, 1), w_spec(C, 1), w_spec(C, 1), w_spec(C, 1), w_spec(C, 1),
        ],
        out_specs=pl.BlockSpec(
            (1, C, _RB, W),
            lambda n, t: (n, 0, jnp.where(t < nt, 0, t - nt), 0)),
        scratch_shapes=[
            pltpu.VMEM((C, H, W), jnp.float32),
            pltpu.VMEM((Ch, H * W), jnp.float32),
            pltpu.VMEM((Ch, H * W), jnp.float32),
            pltpu.VMEM((C, _RH, W), jnp.float32),
            pltpu.VMEM((Ch, _RH, W), jnp.float32),
            pltpu.VMEM((Ch, _RH, W), jnp.float32),
            pltpu.VMEM((C, _RH, W), jnp.float32),
            pltpu.VMEM((C, W), jnp.float32),
            pltpu.VMEM((C, W), jnp.float32),
            pltpu.VMEM((C, W), jnp.float32),
            pltpu.VMEM((_NJ * S * _RH, W), jnp.float32),
            pltpu.VMEM((_NJ * S * _RH, W), jnp.float32),
            pltpu.VMEM((C * _RH, W), jnp.float32),
        ],
        compiler_params=pltpu.CompilerParams(
            dimension_semantics=("parallel", "arbitrary")),
    )(bev, sem, com, wsT, wcT, wabT, wasT, wacT,
      col(b_adp_sem), col(b_adp_com),
      col(b_att_bev), col(b_att_sem), col(b_att_com))

    return out


# speedup vs baseline: 3.2762x; 1.0135x over previous
import jax
import jax.numpy as jnp
from jax.experimental import pallas as pl
from jax.experimental.pallas import tpu as pltpu

_RH = 8
_RB = 32
_NJ = _RB // _RH


def _fused_kernel(bev_ref, sem_ref, com_ref,
                  wsT_ref, wcT_ref, wabT_ref, wasT_ref, wacT_ref,
                  bs_ref, bc_ref, bab_ref, bas_ref, bac_ref,
                  o_ref,
                  cbev_ref, ccat_ref,
                  accb_ref, accs_ref, accc_ref,
                  gw_ref, gb_ref, k_ref,
                  tcat_ref, sout_ref):
    t = pl.program_id(1)
    nt = pl.num_programs(1) // 2
    C, RH, W = accb_ref.shape
    Ch = accs_ref.shape[0]
    S = 2 * Ch + _RH
    S8 = S * _RH
    CW = _RB * W

    @pl.when(t < nt)
    def _phase_a():
        bev_v = bev_ref[0]
        sem_v = sem_ref[0]
        com_v = com_ref[0]

        cbev_ref[:, pl.ds(t * _RB, _RB), :] = bev_v
        for c in range(Ch):
            for j in range(_NJ):
                tcat_ref[j * S8 + c: j * S8 + c + S8: S, :] = \
                    sem_v[c, j * _RH:(j + 1) * _RH, :]
                tcat_ref[j * S8 + Ch + c: j * S8 + Ch + c + S8: S, :] = \
                    com_v[c, j * _RH:(j + 1) * _RH, :]
        cat = jnp.concatenate(
            [tcat_ref[pl.ds(j * S8 + S * s, 2 * Ch), :]
             for j in range(_NJ) for s in range(_RH)], axis=1)
        ccat_ref[:, pl.ds(t * CW, CW)] = cat

        def tilesum(v):
            return sum(v[:, j * _RH:(j + 1) * _RH, :] for j in range(_NJ))

        @pl.when(t == 0)
        def _():
            accb_ref[...] = tilesum(bev_v)
            accs_ref[...] = tilesum(sem_v)
            accc_ref[...] = tilesum(com_v)

        @pl.when(t != 0)
        def _():
            accb_ref[...] += tilesum(bev_v)
            accs_ref[...] += tilesum(sem_v)
            accc_ref[...] += tilesum(com_v)

    @pl.when(t == nt)
    def _gates():
        inv_hw = jnp.float32(1.0 / (nt * _RB * W))

        def fold(acc_ref, n_ch):
            flat = acc_ref[...].reshape(n_ch * RH, W)
            r = jax.lax.broadcasted_iota(jnp.int32, (n_ch, n_ch * RH), 0)
            q = jax.lax.broadcasted_iota(jnp.int32, (n_ch, n_ch * RH), 1)
            sel = (q // RH == r).astype(jnp.float32)
            per_w = jnp.dot(sel, flat, preferred_element_type=jnp.float32)
            return jnp.sum(per_w, axis=1, keepdims=True)

        pb = fold(accb_ref, C) * inv_hw
        ps = fold(accs_ref, Ch) * inv_hw
        pc = fold(accc_ref, Ch) * inv_hw
        ps_a = jnp.dot(wsT_ref[...], ps, preferred_element_type=jnp.float32) + bs_ref[...]
        pc_a = jnp.dot(wcT_ref[...], pc, preferred_element_type=jnp.float32) + bc_ref[...]
        a_b = jax.nn.sigmoid(
            jnp.dot(wabT_ref[...], pb, preferred_element_type=jnp.float32) + bab_ref[...])
        a_s = jax.nn.sigmoid(
            jnp.dot(wasT_ref[...], ps_a, preferred_element_type=jnp.float32) + bas_ref[...])
        a_c = jax.nn.sigmoid(
            jnp.dot(wacT_ref[...], pc_a, preferred_element_type=jnp.float32) + bac_ref[...])
        konst = bs_ref[...] * a_s + bc_ref[...] * a_c

        gw_ref[:, pl.ds(0, Ch)] = wsT_ref[...] * jnp.broadcast_to(a_s, (C, Ch))
        gw_ref[:, pl.ds(Ch, Ch)] = wcT_ref[...] * jnp.broadcast_to(a_c, (C, Ch))
        k_ref[...] = jnp.broadcast_to(konst, (C, W))
        gb_w = jnp.broadcast_to(a_b, (C, W))
        for s in range(_RH):
            sout_ref[s: s + _RH * C: _RH, :] = gb_w
        gb_ref[...] = sout_ref[...].reshape(C, _RH, W)

    @pl.when(t >= nt)
    def _phase_b():
        tb = t - nt
        fused = jnp.dot(gw_ref[...], ccat_ref[:, pl.ds(tb * CW, CW)],
                        preferred_element_type=jnp.float32)
        kk = k_ref[...]
        gbr = gb_ref[...]
        for j in range(_NJ):
            for s in range(_RH):
                k_col = (j * _RH + s) * W
                sout_ref[s: s + _RH * C: _RH, :] = fused[:, k_col:k_col + W] + kk
            o_ref[0, :, j * _RH:(j + 1) * _RH, :] = (
                sout_ref[...].reshape(C, _RH, W)
                + cbev_ref[:, pl.ds(tb * _RB + j * _RH, _RH), :] * gbr)


def kernel(bev, sem, com,
           w_adp_sem, b_adp_sem, w_adp_com, b_adp_com,
           w_att_bev, b_att_bev, w_att_sem, b_att_sem,
           w_att_com, b_att_com):
    N, C, H, W = bev.shape
    Ch = sem.shape[1]
    nt = H // _RB
    S = 2 * Ch + _RH

    wsT = w_adp_sem.T
    wcT = w_adp_com.T
    wabT = w_att_bev.T
    wasT = w_att_sem.T
    wacT = w_att_com.T
    col = lambda b: b.reshape(C, 1)

    w_spec = lambda r, c_: pl.BlockSpec((r, c_), lambda n, t: (0, 0))
    last = nt - 1

    out = pl.pallas_call(
        _fused_kernel,
        out_shape=jax.ShapeDtypeStruct((N, C, H, W), jnp.float32),
        grid=(N, 2 * nt),
        in_specs=[
            pl.BlockSpec((1, C, _RB, W),
                         lambda n, t: (n, 0, jnp.where(t < nt, t, last), 0)),
            pl.BlockSpec((1, Ch, _RB, W),
                         lambda n, t: (n, 0, jnp.where(t < nt, t, last), 0)),
            pl.BlockSpec((1, Ch, _RB, W),
                         lambda n, t: (n, 0, jnp.where(t < nt, t, last), 0)),
            w_spec(C, Ch), w_spec(C, Ch),
            w_spec(C, C), w_spec(C, C), w_spec(C, C),
            w_spec(C, 1), w_spec(C, 1), w_spec(C, 1), w_spec(C, 1), w_spec(C, 1),
        ],
        out_specs=pl.BlockSpec(
            (1, C, _RB, W),
            lambda n, t: (n, 0, jnp.where(t < nt, 0, t - nt), 0)),
        scratch_shapes=[
            pltpu.VMEM((C, H, W), jnp.float32),
            pltpu.VMEM((2 * Ch, H * W), jnp.float32),
            pltpu.VMEM((C, _RH, W), jnp.float32),
            pltpu.VMEM((Ch, _RH, W), jnp.float32),
            pltpu.VMEM((Ch, _RH, W), jnp.float32),
            pltpu.VMEM((C, 2 * Ch), jnp.float32),
            pltpu.VMEM((C, _RH, W), jnp.float32),
            pltpu.VMEM((C, W), jnp.float32),
            pltpu.VMEM((_NJ * S * _RH, W), jnp.float32),
            pltpu.VMEM((C * _RH, W), jnp.float32),
        ],
        compiler_params=pltpu.CompilerParams(
            dimension_semantics=("parallel", "arbitrary")),
    )(bev, sem, com, wsT, wcT, wabT, wasT, wacT,
      col(b_adp_sem), col(b_adp_com),
      col(b_att_bev), col(b_att_sem), col(b_att_com))

    return out


# speedup vs baseline: 3.8392x; 1.1718x over previous
import jax
import jax.numpy as jnp
from jax.experimental import pallas as pl
from jax.experimental.pallas import tpu as pltpu

_RH = 8
_RB = 32
_NJ = _RB // _RH


def _fused_kernel(bev_ref, sem_ref, com_ref,
                  wsT_ref, wcT_ref, wabT_ref, wasT_ref, wacT_ref,
                  bs_ref, bc_ref, bab_ref, bas_ref, bac_ref,
                  o_ref,
                  cbev_ref, ccat_ref,
                  accb_ref, accs_ref, accc_ref,
                  gw_ref, gb_ref, k_ref,
                  tcat_ref, sout_ref):
    n = pl.program_id(0)
    t = pl.program_id(1)
    nt = pl.num_programs(1)
    n_batches = pl.num_programs(0) - 1
    C, RH, W = gb_ref.shape
    Ch = accs_ref.shape[1]
    S = 2 * Ch + _RH
    S8 = S * _RH
    CW = _RB * W
    slot = jax.lax.rem(n, 2)
    pslot = jax.lax.rem(n + 1, 2)

    @pl.when(n < n_batches)
    def _phase_a():
        bev_v = bev_ref[0]
        sem_v = sem_ref[0]
        com_v = com_ref[0]

        cbev_ref[slot, :, pl.ds(t * _RB, _RB), :] = bev_v
        for c in range(Ch):
            for j in range(_NJ):
                tcat_ref[j * S8 + c: j * S8 + c + S8: S, :] = \
                    sem_v[c, j * _RH:(j + 1) * _RH, :]
                tcat_ref[j * S8 + Ch + c: j * S8 + Ch + c + S8: S, :] = \
                    com_v[c, j * _RH:(j + 1) * _RH, :]
        cat = jnp.concatenate(
            [tcat_ref[pl.ds(j * S8 + S * s, 2 * Ch), :]
             for j in range(_NJ) for s in range(_RH)], axis=1)
        ccat_ref[slot, :, pl.ds(t * CW, CW)] = cat

        def tilesum(v):
            return sum(v[:, j * _RH:(j + 1) * _RH, :] for j in range(_NJ))

        @pl.when(t == 0)
        def _():
            accb_ref[slot] = tilesum(bev_v)
            accs_ref[slot] = tilesum(sem_v)
            accc_ref[slot] = tilesum(com_v)

        @pl.when(t != 0)
        def _():
            accb_ref[slot] += tilesum(bev_v)
            accs_ref[slot] += tilesum(sem_v)
            accc_ref[slot] += tilesum(com_v)

    @pl.when((n >= 1) & (t == 0))
    def _gates():
        inv_hw = jnp.float32(1.0 / (nt * _RB * W))

        def fold(acc_ref, n_ch):
            flat = acc_ref[pslot].reshape(n_ch * RH, W)
            r = jax.lax.broadcasted_iota(jnp.int32, (n_ch, n_ch * RH), 0)
            q = jax.lax.broadcasted_iota(jnp.int32, (n_ch, n_ch * RH), 1)
            sel = (q // RH == r).astype(jnp.float32)
            per_w = jnp.dot(sel, flat, preferred_element_type=jnp.float32)
            return jnp.sum(per_w, axis=1, keepdims=True)

        pb = fold(accb_ref, C) * inv_hw
        ps = fold(accs_ref, Ch) * inv_hw
        pc = fold(accc_ref, Ch) * inv_hw
        ps_a = jnp.dot(wsT_ref[...], ps, preferred_element_type=jnp.float32) + bs_ref[...]
        pc_a = jnp.dot(wcT_ref[...], pc, preferred_element_type=jnp.float32) + bc_ref[...]
        a_b = jax.nn.sigmoid(
            jnp.dot(wabT_ref[...], pb, preferred_element_type=jnp.float32) + bab_ref[...])
        a_s = jax.nn.sigmoid(
            jnp.dot(wasT_ref[...], ps_a, preferred_element_type=jnp.float32) + bas_ref[...])
        a_c = jax.nn.sigmoid(
            jnp.dot(wacT_ref[...], pc_a, preferred_element_type=jnp.float32) + bac_ref[...])
        konst = bs_ref[...] * a_s + bc_ref[...] * a_c

        gw_ref[:, pl.ds(0, Ch)] = wsT_ref[...] * jnp.broadcast_to(a_s, (C, Ch))
        gw_ref[:, pl.ds(Ch, Ch)] = wcT_ref[...] * jnp.broadcast_to(a_c, (C, Ch))
        k_ref[...] = jnp.broadcast_to(konst, (C, W))
        gb_w = jnp.broadcast_to(a_b, (C, W))
        for s in range(_RH):
            sout_ref[s: s + _RH * C: _RH, :] = gb_w
        gb_ref[...] = sout_ref[...].reshape(C, _RH, W)

    @pl.when(n >= 1)
    def _phase_b():
        fused = jnp.dot(gw_ref[...], ccat_ref[pslot, :, pl.ds(t * CW, CW)],
                        preferred_element_type=jnp.float32)
        kk = k_ref[...]
        gbr = gb_ref[...]
        for j in range(_NJ):
            for s in range(_RH):
                k_col = (j * _RH + s) * W
                sout_ref[s: s + _RH * C: _RH, :] = fused[:, k_col:k_col + W] + kk
            o_ref[0, :, j * _RH:(j + 1) * _RH, :] = (
                sout_ref[...].reshape(C, _RH, W)
                + cbev_ref[pslot, :, pl.ds(t * _RB + j * _RH, _RH), :] * gbr)


def kernel(bev, sem, com,
           w_adp_sem, b_adp_sem, w_adp_com, b_adp_com,
           w_att_bev, b_att_bev, w_att_sem, b_att_sem,
           w_att_com, b_att_com):
    N, C, H, W = bev.shape
    Ch = sem.shape[1]
    nt = H // _RB
    S = 2 * Ch + _RH

    wsT = w_adp_sem.T
    wcT = w_adp_com.T
    wabT = w_att_bev.T
    wasT = w_att_sem.T
    wacT = w_att_com.T
    col = lambda b: b.reshape(C, 1)

    w_spec = lambda r, c_: pl.BlockSpec((r, c_), lambda n, t: (0, 0))
    last = nt - 1

    def in_map(n, t):
        return (jnp.minimum(n, N - 1), 0, jnp.where(n < N, t, last), 0)

    out = pl.pallas_call(
        _fused_kernel,
        out_shape=jax.ShapeDtypeStruct((N, C, H, W), jnp.float32),
        grid=(N + 1, nt),
        in_specs=[
            pl.BlockSpec((1, C, _RB, W), in_map),
            pl.BlockSpec((1, Ch, _RB, W), in_map),
            pl.BlockSpec((1, Ch, _RB, W), in_map),
            w_spec(C, Ch), w_spec(C, Ch),
            w_spec(C, C), w_spec(C, C), w_spec(C, C),
            w_spec(C, 1), w_spec(C, 1), w_spec(C, 1), w_spec(C, 1), w_spec(C, 1),
        ],
        out_specs=pl.BlockSpec(
            (1, C, _RB, W),
            lambda n, t: (jnp.maximum(n - 1, 0), 0,
                          jnp.where(n >= 1, t, 0), 0)),
        scratch_shapes=[
            pltpu.VMEM((2, C, H, W), jnp.float32),
            pltpu.VMEM((2, 2 * Ch, H * W), jnp.float32),
            pltpu.VMEM((2, C, _RH, W), jnp.float32),
            pltpu.VMEM((2, Ch, _RH, W), jnp.float32),
            pltpu.VMEM((2, Ch, _RH, W), jnp.float32),
            pltpu.VMEM((C, 2 * Ch), jnp.float32),
            pltpu.VMEM((C, _RH, W), jnp.float32),
            pltpu.VMEM((C, W), jnp.float32),
            pltpu.VMEM((_NJ * S * _RH, W), jnp.float32),
            pltpu.VMEM((C * _RH, W), jnp.float32),
        ],
        compiler_params=pltpu.CompilerParams(
            dimension_semantics=("arbitrary", "arbitrary")),
    )(bev, sem, com, wsT, wcT, wabT, wasT, wacT,
      col(b_adp_sem), col(b_adp_com),
      col(b_att_bev), col(b_att_sem), col(b_att_com))

    return out


# speedup vs baseline: 3.9095x; 1.0183x over previous
import jax
import jax.numpy as jnp
from jax.experimental import pallas as pl
from jax.experimental.pallas import tpu as pltpu

_RH = 8
_RB = 32
_NJ = _RB // _RH


def _fused_kernel(bev_ref, sem_ref, com_ref,
                  wsT_ref, wcT_ref, wabT_ref, wasT_ref, wacT_ref,
                  bs_ref, bc_ref, bab_ref, bas_ref, bac_ref,
                  o_ref,
                  cbev_ref, ccat_ref,
                  accb_ref, accs_ref, accc_ref,
                  gw_ref, gb_ref, k_ref, sout_ref):
    n = pl.program_id(0)
    t = pl.program_id(1)
    nt = pl.num_programs(1)
    n_batches = pl.num_programs(0) - 1
    C, RH, W = gb_ref.shape
    Ch = accs_ref.shape[1]
    S = 2 * Ch + _RH
    S8 = S * _RH
    CW = _RB * W
    slot = jax.lax.rem(n, 2)
    pslot = jax.lax.rem(n + 1, 2)

    @pl.when(n < n_batches)
    def _phase_a():
        bev_v = bev_ref[0]
        sem_v = sem_ref[0]
        com_v = com_ref[0]

        cbev_ref[slot, :, pl.ds(t * _RB, _RB), :] = bev_v
        for c in range(Ch):
            for j in range(_NJ):
                ccat_ref[slot, t, j * S8 + c: j * S8 + c + S8: S, :] = \
                    sem_v[c, j * _RH:(j + 1) * _RH, :]
                ccat_ref[slot, t, j * S8 + Ch + c: j * S8 + Ch + c + S8: S, :] = \
                    com_v[c, j * _RH:(j + 1) * _RH, :]


        def tilesum(v):
            return sum(v[:, j * _RH:(j + 1) * _RH, :] for j in range(_NJ))

        @pl.when(t == 0)
        def _():
            accb_ref[slot] = tilesum(bev_v)
            accs_ref[slot] = tilesum(sem_v)
            accc_ref[slot] = tilesum(com_v)

        @pl.when(t != 0)
        def _():
            accb_ref[slot] += tilesum(bev_v)
            accs_ref[slot] += tilesum(sem_v)
            accc_ref[slot] += tilesum(com_v)

    @pl.when((n >= 1) & (t == 0))
    def _gates():
        inv_hw = jnp.float32(1.0 / (nt * _RB * W))

        def fold(acc_ref, n_ch):
            flat = acc_ref[pslot].reshape(n_ch * RH, W)
            r = jax.lax.broadcasted_iota(jnp.int32, (n_ch, n_ch * RH), 0)
            q = jax.lax.broadcasted_iota(jnp.int32, (n_ch, n_ch * RH), 1)
            sel = (q // RH == r).astype(jnp.float32)
            per_w = jnp.dot(sel, flat, preferred_element_type=jnp.float32)
            return jnp.sum(per_w, axis=1, keepdims=True)

        pb = fold(accb_ref, C) * inv_hw
        ps = fold(accs_ref, Ch) * inv_hw
        pc = fold(accc_ref, Ch) * inv_hw
        ps_a = jnp.dot(wsT_ref[...], ps, preferred_element_type=jnp.float32) + bs_ref[...]
        pc_a = jnp.dot(wcT_ref[...], pc, preferred_element_type=jnp.float32) + bc_ref[...]
        a_b = jax.nn.sigmoid(
            jnp.dot(wabT_ref[...], pb, preferred_element_type=jnp.float32) + bab_ref[...])
        a_s = jax.nn.sigmoid(
            jnp.dot(wasT_ref[...], ps_a, preferred_element_type=jnp.float32) + bas_ref[...])
        a_c = jax.nn.sigmoid(
            jnp.dot(wacT_ref[...], pc_a, preferred_element_type=jnp.float32) + bac_ref[...])
        konst = bs_ref[...] * a_s + bc_ref[...] * a_c

        gw_ref[:, pl.ds(0, Ch)] = wsT_ref[...] * jnp.broadcast_to(a_s, (C, Ch))
        gw_ref[:, pl.ds(Ch, Ch)] = wcT_ref[...] * jnp.broadcast_to(a_c, (C, Ch))
        k_ref[...] = jnp.broadcast_to(konst, (C, W))
        gb_w = jnp.broadcast_to(a_b, (C, W))
        for s in range(_RH):
            sout_ref[s: s + _RH * C: _RH, :] = gb_w
        gb_ref[...] = sout_ref[...].reshape(C, _RH, W)

    @pl.when(n >= 1)
    def _phase_b():
        cat = jnp.concatenate(
            [ccat_ref[pslot, t, pl.ds(j * S8 + S * s, 2 * Ch), :]
             for j in range(_NJ) for s in range(_RH)], axis=1)
        fused = jnp.dot(gw_ref[...], cat,
                        preferred_element_type=jnp.float32)
        kk = k_ref[...]
        gbr = gb_ref[...]
        for j in range(_NJ):
            for s in range(_RH):
                k_col = (j * _RH + s) * W
                sout_ref[s: s + _RH * C: _RH, :] = fused[:, k_col:k_col + W] + kk
            o_ref[0, :, j * _RH:(j + 1) * _RH, :] = (
                sout_ref[...].reshape(C, _RH, W)
                + cbev_ref[pslot, :, pl.ds(t * _RB + j * _RH, _RH), :] * gbr)


def kernel(bev, sem, com,
           w_adp_sem, b_adp_sem, w_adp_com, b_adp_com,
           w_att_bev, b_att_bev, w_att_sem, b_att_sem,
           w_att_com, b_att_com):
    N, C, H, W = bev.shape
    Ch = sem.shape[1]
    nt = H // _RB
    S = 2 * Ch + _RH

    wsT = w_adp_sem.T
    wcT = w_adp_com.T
    wabT = w_att_bev.T
    wasT = w_att_sem.T
    wacT = w_att_com.T
    col = lambda b: b.reshape(C, 1)

    w_spec = lambda r, c_: pl.BlockSpec((r, c_), lambda n, t: (0, 0))
    last = nt - 1

    def in_map(n, t):
        return (jnp.minimum(n, N - 1), 0, jnp.where(n < N, t, last), 0)

    out = pl.pallas_call(
        _fused_kernel,
        out_shape=jax.ShapeDtypeStruct((N, C, H, W), jnp.float32),
        grid=(N + 1, nt),
        in_specs=[
            pl.BlockSpec((1, C, _RB, W), in_map),
            pl.BlockSpec((1, Ch, _RB, W), in_map),
            pl.BlockSpec((1, Ch, _RB, W), in_map),
            w_spec(C, Ch), w_spec(C, Ch),
            w_spec(C, C), w_spec(C, C), w_spec(C, C),
            w_spec(C, 1), w_spec(C, 1), w_spec(C, 1), w_spec(C, 1), w_spec(C, 1),
        ],
        out_specs=pl.BlockSpec(
            (1, C, _RB, W),
            lambda n, t: (jnp.maximum(n - 1, 0), 0,
                          jnp.where(n >= 1, t, 0), 0)),
        scratch_shapes=[
            pltpu.VMEM((2, C, H, W), jnp.float32),
            pltpu.VMEM((2, nt, _NJ * S * _RH, W), jnp.float32),
            pltpu.VMEM((2, C, _RH, W), jnp.float32),
            pltpu.VMEM((2, Ch, _RH, W), jnp.float32),
            pltpu.VMEM((2, Ch, _RH, W), jnp.float32),
            pltpu.VMEM((C, 2 * Ch), jnp.float32),
            pltpu.VMEM((C, _RH, W), jnp.float32),
            pltpu.VMEM((C, W), jnp.float32),
            pltpu.VMEM((C * _RH, W), jnp.float32),
        ],
        compiler_params=pltpu.CompilerParams(
            dimension_semantics=("arbitrary", "arbitrary")),
    )(bev, sem, com, wsT, wcT, wabT, wasT, wacT,
      col(b_adp_sem), col(b_adp_com),
      col(b_att_bev), col(b_att_sem), col(b_att_com))

    return out
